# Initial kernel scaffold; baseline (speedup 1.0000x reference)
import jax
import jax.numpy as jnp
from jax.experimental import pallas as pl


def kernel(node_features, ensemble_index, W0, b0, W1, b1, W2, b2):
    raise NotImplementedError("write your pallas kernel here")


# speedup vs baseline: 2.4892x; 2.4892x over previous
import jax
import jax.numpy as jnp
from jax import lax
from jax.experimental import pallas as pl
from jax.experimental.pallas import tpu as pltpu
from jax.experimental.pallas import tpu_sc as plsc

N = 320000
D = 128
H = 64
S = 10000
SPAD = 10240

NC = 2
NS = 16
NW = NC * NS
ROWS_PER_TILE = N // NW
CHUNK = 80
NCHUNK = ROWS_PER_TILE // CHUNK
MLP_B = 3200

DEN_B = 3200
DEN_W = 128
DEN_PAD = S + DEN_W


def _mlp_body(f_ref, w0_ref, b0_ref, w1_ref, b1_ref, w2_ref, b2_ref, o_ref):
    x = f_ref[...]
    h = jnp.dot(x, w0_ref[...], preferred_element_type=jnp.float32) + b0_ref[...]
    h = h * lax.logistic(h)
    h = jnp.dot(h, w1_ref[...], preferred_element_type=jnp.float32) + b1_ref[...]
    h = h * lax.logistic(h)
    w = jnp.dot(h, w2_ref[...], preferred_element_type=jnp.float32) + b2_ref[...]
    o_ref[...] = w


def _mlp_weights(node_features, W0, b0r, W1, b1r, w2r, b2r):
    return pl.pallas_call(
        _mlp_body,
        grid=(N // MLP_B,),
        in_specs=[
            pl.BlockSpec((MLP_B, D), lambda i: (i, 0)),
            pl.BlockSpec((D, H), lambda i: (0, 0)),
            pl.BlockSpec((1, H), lambda i: (0, 0)),
            pl.BlockSpec((H, H), lambda i: (0, 0)),
            pl.BlockSpec((1, H), lambda i: (0, 0)),
            pl.BlockSpec((H, 1), lambda i: (0, 0)),
            pl.BlockSpec((1, 1), lambda i: (0, 0)),
        ],
        out_specs=pl.BlockSpec((MLP_B, 1), lambda i: (i, 0)),
        out_shape=jax.ShapeDtypeStruct((N, 1), jnp.float32),
    )(node_features, W0, b0r, W1, b1r, w2r, b2r)


def _sc_body(f_hbm, w_hbm, i_hbm, out_hbm, fbuf, wbuf, ibuf, wfbuf, acc):
    cid = lax.axis_index("c")
    sid = lax.axis_index("s")
    wid = cid * NS + sid
    base_row = wid * ROWS_PER_TILE

    def _zero_row(r, carry):
        for c in range(D // 16):
            wfbuf[r, pl.ds(c * 16, 16)] = jnp.zeros((16,), jnp.float32)
        return carry

    lax.fori_loop(0, CHUNK, _zero_row, 0)
    for k in range(SPAD // NS // CHUNK):
        pltpu.sync_copy(
            wfbuf, acc.at[pl.ds(sid * (SPAD // NS) + k * CHUNK, CHUNK), :]
        )
    plsc.subcore_barrier()

    def _chunk(k, carry):
        r0 = base_row + k * CHUNK
        pltpu.sync_copy(f_hbm.at[pl.ds(r0, CHUNK), :], fbuf)
        pltpu.sync_copy(w_hbm.at[pl.ds(r0, CHUNK)], wbuf)
        pltpu.sync_copy(i_hbm.at[pl.ds(r0, CHUNK)], ibuf)

        def _group(g, c2):
            w16 = wbuf[pl.ds(g * 16, 16)]
            for r in range(16):
                row = g * 16 + r
                wbc = jnp.full((16,), w16[r], jnp.float32)
                for c in range(D // 16):
                    wfbuf[row, pl.ds(c * 16, 16)] = (
                        fbuf[row, pl.ds(c * 16, 16)] * wbc
                    )
            return c2

        lax.fori_loop(0, CHUNK // 16, _group, 0)
        pltpu.sync_copy(wfbuf, acc.at[ibuf], add=True)
        return carry

    lax.fori_loop(0, NCHUNK, _chunk, 0)
    plsc.subcore_barrier()

    rows = SPAD // NS
    pltpu.sync_copy(
        acc.at[pl.ds(sid * rows, rows), :],
        out_hbm.at[cid, pl.ds(sid * rows, rows), :],
    )


def _sc_scatter(node_features, wflat, idx):
    mesh = plsc.VectorSubcoreMesh(core_axis_name="c", subcore_axis_name="s")
    run = pl.kernel(
        _sc_body,
        out_type=jax.ShapeDtypeStruct((NC, SPAD, D), jnp.float32),
        mesh=mesh,
        scratch_types=[
            pltpu.VMEM((CHUNK, D), jnp.float32),
            pltpu.VMEM((CHUNK,), jnp.float32),
            pltpu.VMEM((CHUNK,), jnp.int32),
            pltpu.VMEM((CHUNK, D), jnp.float32),
            pltpu.VMEM_SHARED((SPAD, D), jnp.float32),
        ],
    )
    return run(node_features, wflat, idx)


def _den_body(bases_ref, nch_ref, idx_ref, w_ref, den_ref):
    pi = pl.program_id(0)

    @pl.when(pi == 0)
    def _():
        den_ref[...] = jnp.zeros_like(den_ref)

    base = bases_ref[pi]
    nch = nch_ref[pi]
    rel = idx_ref[0, 0, :] - base
    w = w_ref[...]

    def _win(k, carry):
        lo = k * DEN_W
        cols = lax.broadcasted_iota(jnp.int32, (DEN_B, DEN_W), 1)
        onehot = (rel[:, None] - lo == cols).astype(jnp.float32)
        contrib = lax.dot_general(
            onehot, w, (((0,), (0,)), ((), ())),
            preferred_element_type=jnp.float32,
            precision=lax.Precision.HIGHEST,
        )
        den_ref[pl.ds(base + lo, DEN_W), :] += contrib
        return carry

    lax.fori_loop(0, nch, _win, 0)


def _denominator(wflat, idx, bases, nch):
    idx3 = idx.reshape(N // DEN_B, 1, DEN_B)
    grid_spec = pltpu.PrefetchScalarGridSpec(
        num_scalar_prefetch=2,
        grid=(N // DEN_B,),
        in_specs=[
            pl.BlockSpec((1, 1, DEN_B), lambda i, b, n: (i, 0, 0)),
            pl.BlockSpec((DEN_B, 1), lambda i, b, n: (i, 0)),
        ],
        out_specs=pl.BlockSpec((DEN_PAD, 1), lambda i, b, n: (0, 0)),
    )
    return pl.pallas_call(
        _den_body,
        grid_spec=grid_spec,
        out_shape=jax.ShapeDtypeStruct((DEN_PAD, 1), jnp.float32),
    )(bases, nch, idx3, wflat.reshape(N, 1))


def _combine_body(p_ref, den_ref, o_ref):
    s = p_ref[0] + p_ref[1]
    den = den_ref[...]
    den = jnp.where(den == 0.0, 1.0, den)
    o_ref[...] = s / den


def _combine(parts, den):
    rb = 400
    return pl.pallas_call(
        _combine_body,
        grid=(S // rb,),
        in_specs=[
            pl.BlockSpec((NC, rb, D), lambda i: (0, i, 0)),
            pl.BlockSpec((rb, 1), lambda i: (i, 0)),
        ],
        out_specs=pl.BlockSpec((rb, D), lambda i: (i, 0)),
        out_shape=jax.ShapeDtypeStruct((S, D), jnp.float32),
    )(parts, den)


def kernel(node_features, ensemble_index, W0, b0, W1, b1, W2, b2):
    idx = ensemble_index.astype(jnp.int32)
    b0r = b0.reshape(1, H)
    b1r = b1.reshape(1, H)
    b2r = b2.reshape(1, 1)
    w2d = _mlp_weights(node_features, W0, b0r, W1, b1r, W2, b2r)
    wflat = w2d.reshape(N)
    bases = idx[:: DEN_B]
    lasts = idx[DEN_B - 1 :: DEN_B]
    nch = (lasts - bases) // DEN_W + 1
    parts = _sc_scatter(node_features, wflat, idx)
    den = _denominator(wflat, idx, bases, nch)
    return _combine(parts, den)


# speedup vs baseline: 3.1787x; 1.2770x over previous
import jax
import jax.numpy as jnp
from jax import lax
from jax.experimental import pallas as pl
from jax.experimental.pallas import tpu as pltpu
from jax.experimental.pallas import tpu_sc as plsc

N = 320000
D = 128
H = 64
S = 10000
SPAD = 10240

NC = 2
NS = 16
NW = NC * NS
ROWS_PER_TILE = N // NW
CHUNK = 80
NCHUNK = ROWS_PER_TILE // CHUNK
MLP_B = 3200

DEN_B = 3200
DEN_W = 128
DEN_PAD = S + DEN_W


def _mlp_body(bases_ref, nch_ref, f_ref, w0_ref, b0_ref, w1_ref, b1_ref,
              w2_ref, b2_ref, idx_ref, o_ref, den_ref):
    pi = pl.program_id(0)

    @pl.when(pi == 0)
    def _():
        den_ref[...] = jnp.zeros_like(den_ref)

    x = f_ref[...]
    h = jnp.dot(x, w0_ref[...], preferred_element_type=jnp.float32) + b0_ref[...]
    h = h * lax.logistic(h)
    h = jnp.dot(h, w1_ref[...], preferred_element_type=jnp.float32) + b1_ref[...]
    h = h * lax.logistic(h)
    w = jnp.dot(h, w2_ref[...], preferred_element_type=jnp.float32) + b2_ref[...]
    o_ref[...] = w

    base = bases_ref[pi]
    nch = nch_ref[pi]
    rel = (idx_ref[0, 0, :] - base)[None, :]

    def _win(k, carry):
        lo = k * DEN_W
        rows = lax.broadcasted_iota(jnp.int32, (DEN_W, MLP_B), 0)
        onehot_t = (rel - lo == rows).astype(jnp.float32)
        contrib = jnp.dot(
            onehot_t, w,
            preferred_element_type=jnp.float32,
            precision=lax.Precision.HIGHEST,
        )
        den_ref[pl.ds(base + lo, DEN_W), :] += contrib
        return carry

    lax.fori_loop(0, nch, _win, 0)


def _mlp_weights(node_features, W0, b0r, W1, b1r, W2, b2r, idx3, bases, nch):
    grid_spec = pltpu.PrefetchScalarGridSpec(
        num_scalar_prefetch=2,
        grid=(N // MLP_B,),
        in_specs=[
            pl.BlockSpec((MLP_B, D), lambda i, b, n: (i, 0)),
            pl.BlockSpec((D, H), lambda i, b, n: (0, 0)),
            pl.BlockSpec((1, H), lambda i, b, n: (0, 0)),
            pl.BlockSpec((H, H), lambda i, b, n: (0, 0)),
            pl.BlockSpec((1, H), lambda i, b, n: (0, 0)),
            pl.BlockSpec((H, 1), lambda i, b, n: (0, 0)),
            pl.BlockSpec((1, 1), lambda i, b, n: (0, 0)),
            pl.BlockSpec((1, 1, MLP_B), lambda i, b, n: (i, 0, 0)),
        ],
        out_specs=[
            pl.BlockSpec((MLP_B, 1), lambda i, b, n: (i, 0)),
            pl.BlockSpec((DEN_PAD, 1), lambda i, b, n: (0, 0)),
        ],
    )
    return pl.pallas_call(
        _mlp_body,
        grid_spec=grid_spec,
        out_shape=[
            jax.ShapeDtypeStruct((N, 1), jnp.float32),
            jax.ShapeDtypeStruct((DEN_PAD, 1), jnp.float32),
        ],
    )(bases, nch, node_features, W0, b0r, W1, b1r, W2, b2r, idx3)


def _sc_body(f_hbm, w_hbm, i_hbm, out_hbm,
             fb0, fb1, wb0, wb1, ib0, ib1, wf0, wf1, si0, si1, acc,
             in0, in1, sc0, sc1):
    cid = lax.axis_index("c")
    sid = lax.axis_index("s")
    wid = cid * NS + sid
    base_row = wid * ROWS_PER_TILE
    fb = (fb0, fb1)
    wb = (wb0, wb1)
    ib = (ib0, ib1)
    wf = (wf0, wf1)
    si = (si0, si1)
    sin = (in0, in1)
    ssc = (sc0, sc1)

    def issue_inputs(k, p):
        r0 = base_row + k * CHUNK
        pltpu.async_copy(f_hbm.at[pl.ds(r0, CHUNK), :], fb[p], sin[p])
        pltpu.async_copy(w_hbm.at[pl.ds(r0, CHUNK)], wb[p], sin[p])
        pltpu.async_copy(i_hbm.at[pl.ds(r0, CHUNK)], ib[p], sin[p])

    def wait_inputs(k, p):
        r0 = base_row + k * CHUNK
        pltpu.make_async_copy(f_hbm.at[pl.ds(r0, CHUNK), :], fb[p], sin[p]).wait()
        pltpu.make_async_copy(w_hbm.at[pl.ds(r0, CHUNK)], wb[p], sin[p]).wait()
        pltpu.make_async_copy(i_hbm.at[pl.ds(r0, CHUNK)], ib[p], sin[p]).wait()

    def compute(p):
        def _group(g, c2):
            w16 = wb[p][pl.ds(g * 16, 16)]
            si[p][pl.ds(g * 16, 16)] = ib[p][pl.ds(g * 16, 16)]
            for r in range(16):
                row = g * 16 + r
                wbc = jnp.full((16,), w16[r], jnp.float32)
                for c in range(D // 16):
                    wf[p][row, pl.ds(c * 16, 16)] = (
                        fb[p][row, pl.ds(c * 16, 16)] * wbc
                    )
            return c2

        lax.fori_loop(0, CHUNK // 16, _group, 0)

    def issue_scatter(p):
        pltpu.async_copy(wf[p], acc.at[si[p]], ssc[p], add=True)

    def wait_scatter(p):
        pltpu.make_async_copy(wf[p], acc.at[si[p]], ssc[p]).wait()

    issue_inputs(0, 0)

    def _zero_row(r, carry):
        for c in range(D // 16):
            wf0[r, pl.ds(c * 16, 16)] = jnp.zeros((16,), jnp.float32)
        return carry

    lax.fori_loop(0, CHUNK, _zero_row, 0)
    for k in range(SPAD // NS // CHUNK):
        pltpu.sync_copy(
            wf0, acc.at[pl.ds(sid * (SPAD // NS) + k * CHUNK, CHUNK), :]
        )
    plsc.subcore_barrier()

    issue_inputs(1, 1)
    wait_inputs(0, 0)
    compute(0)
    issue_scatter(0)
    issue_inputs(2, 0)
    wait_inputs(1, 1)
    compute(1)
    issue_scatter(1)

    def _pair(m, carry):
        k0 = 2 * m
        issue_inputs(k0 + 1, 1)
        wait_inputs(k0, 0)
        wait_scatter(0)
        compute(0)
        issue_scatter(0)
        issue_inputs(k0 + 2, 0)
        wait_inputs(k0 + 1, 1)
        wait_scatter(1)
        compute(1)
        issue_scatter(1)
        return carry

    lax.fori_loop(1, (NCHUNK - 1) // 2, _pair, 0)

    wait_inputs(NCHUNK - 1, 0)
    wait_scatter(0)
    compute(0)
    issue_scatter(0)
    wait_scatter(1)
    wait_scatter(0)
    plsc.subcore_barrier()

    rows = SPAD // NS
    pltpu.sync_copy(
        acc.at[pl.ds(sid * rows, rows), :],
        out_hbm.at[cid, pl.ds(sid * rows, rows), :],
    )


def _sc_scatter(node_features, wflat, idx):
    mesh = plsc.VectorSubcoreMesh(core_axis_name="c", subcore_axis_name="s")
    run = pl.kernel(
        _sc_body,
        out_type=jax.ShapeDtypeStruct((NC, SPAD, D), jnp.float32),
        mesh=mesh,
        scratch_types=[
            pltpu.VMEM((CHUNK, D), jnp.float32),
            pltpu.VMEM((CHUNK, D), jnp.float32),
            pltpu.VMEM((CHUNK,), jnp.float32),
            pltpu.VMEM((CHUNK,), jnp.float32),
            pltpu.VMEM((CHUNK,), jnp.int32),
            pltpu.VMEM((CHUNK,), jnp.int32),
            pltpu.VMEM((CHUNK, D), jnp.float32),
            pltpu.VMEM((CHUNK, D), jnp.float32),
            pltpu.VMEM((CHUNK,), jnp.int32),
            pltpu.VMEM((CHUNK,), jnp.int32),
            pltpu.VMEM_SHARED((SPAD, D), jnp.float32),
            pltpu.SemaphoreType.DMA,
            pltpu.SemaphoreType.DMA,
            pltpu.SemaphoreType.DMA,
            pltpu.SemaphoreType.DMA,
        ],
    )
    return run(node_features, wflat, idx)


def _combine_body(p_ref, den_ref, o_ref):
    s = p_ref[0] + p_ref[1]
    den = den_ref[...]
    den = jnp.where(den == 0.0, 1.0, den)
    o_ref[...] = s / den


def _combine(parts, den):
    rb = 400
    return pl.pallas_call(
        _combine_body,
        grid=(S // rb,),
        in_specs=[
            pl.BlockSpec((NC, rb, D), lambda i: (0, i, 0)),
            pl.BlockSpec((rb, 1), lambda i: (i, 0)),
        ],
        out_specs=pl.BlockSpec((rb, D), lambda i: (i, 0)),
        out_shape=jax.ShapeDtypeStruct((S, D), jnp.float32),
    )(parts, den)


def kernel(node_features, ensemble_index, W0, b0, W1, b1, W2, b2):
    idx = ensemble_index.astype(jnp.int32)
    b0r = b0.reshape(1, H)
    b1r = b1.reshape(1, H)
    b2r = b2.reshape(1, 1)
    bases = idx[::MLP_B]
    lasts = idx[MLP_B - 1 :: MLP_B]
    nch = (lasts - bases) // DEN_W + 1
    idx3 = idx.reshape(N // MLP_B, 1, MLP_B)
    w2d, den = _mlp_weights(node_features, W0, b0r, W1, b1r, W2, b2r,
                            idx3, bases, nch)
    wflat = w2d.reshape(N)
    parts = _sc_scatter(node_features, wflat, idx)
    return _combine(parts, den)


# speedup vs baseline: 3.6987x; 1.1636x over previous
import jax
import jax.numpy as jnp
from jax import lax
from jax.experimental import pallas as pl
from jax.experimental.pallas import tpu as pltpu
from jax.experimental.pallas import tpu_sc as plsc

N = 320000
D = 128
H = 64
S = 10000
SPAD = 10240

NC = 2
NS = 16
NW = NC * NS
ROWS_PER_TILE = N // NW
CHUNK = 80
NCHUNK = ROWS_PER_TILE // CHUNK
MLP_B = 3200

DEN_W = 128
DEN_PAD = S + DEN_W


def _mlp_body(bases_ref, nch_ref, f_ref, w0_ref, b0_ref, w1_ref, b1_ref,
              w2_ref, b2_ref, idx_ref, o_ref, den_ref):
    pi = pl.program_id(0)

    @pl.when(pi == 0)
    def _():
        den_ref[...] = jnp.zeros_like(den_ref)

    x = f_ref[...]
    h = jnp.dot(x, w0_ref[...], preferred_element_type=jnp.float32) + b0_ref[...]
    h = h * lax.logistic(h)
    h = jnp.dot(h, w1_ref[...], preferred_element_type=jnp.float32) + b1_ref[...]
    h = h * lax.logistic(h)
    w = jnp.dot(h, w2_ref[...], preferred_element_type=jnp.float32) + b2_ref[...]
    o_ref[...] = w

    base = bases_ref[pi]
    nch = nch_ref[pi]
    rel = (idx_ref[0, 0, :] - base)[None, :]

    w_hi = w.astype(jnp.bfloat16)
    r1 = w - w_hi.astype(jnp.float32)
    w_mid = r1.astype(jnp.bfloat16)
    w_lo = (r1 - w_mid.astype(jnp.float32)).astype(jnp.bfloat16)

    def _win(k, carry):
        lo = k * DEN_W
        rows = lax.broadcasted_iota(jnp.int32, (DEN_W, MLP_B), 0)
        onehot_t = (rel - lo == rows).astype(jnp.bfloat16)
        contrib = (
            jnp.dot(onehot_t, w_hi, preferred_element_type=jnp.float32)
            + jnp.dot(onehot_t, w_mid, preferred_element_type=jnp.float32)
            + jnp.dot(onehot_t, w_lo, preferred_element_type=jnp.float32)
        )
        den_ref[pl.ds(base + lo, DEN_W), :] += contrib
        return carry

    lax.fori_loop(0, nch, _win, 0)


def _mlp_weights(node_features, W0, b0r, W1, b1r, W2, b2r, idx3, bases, nch):
    grid_spec = pltpu.PrefetchScalarGridSpec(
        num_scalar_prefetch=2,
        grid=(N // MLP_B,),
        in_specs=[
            pl.BlockSpec((MLP_B, D), lambda i, b, n: (i, 0)),
            pl.BlockSpec((D, H), lambda i, b, n: (0, 0)),
            pl.BlockSpec((1, H), lambda i, b, n: (0, 0)),
            pl.BlockSpec((H, H), lambda i, b, n: (0, 0)),
            pl.BlockSpec((1, H), lambda i, b, n: (0, 0)),
            pl.BlockSpec((H, 1), lambda i, b, n: (0, 0)),
            pl.BlockSpec((1, 1), lambda i, b, n: (0, 0)),
            pl.BlockSpec((1, 1, MLP_B), lambda i, b, n: (i, 0, 0)),
        ],
        out_specs=[
            pl.BlockSpec((MLP_B, 1), lambda i, b, n: (i, 0)),
            pl.BlockSpec((DEN_PAD, 1), lambda i, b, n: (0, 0)),
        ],
    )
    return pl.pallas_call(
        _mlp_body,
        grid_spec=grid_spec,
        out_shape=[
            jax.ShapeDtypeStruct((N, 1), jnp.float32),
            jax.ShapeDtypeStruct((DEN_PAD, 1), jnp.float32),
        ],
    )(bases, nch, node_features, W0, b0r, W1, b1r, W2, b2r, idx3)


def _sc_body(f_hbm, w_hbm, i_hbm, out_hbm,
             fb0, fb1, wb0, wb1, ib0, ib1, wf0, wf1, si0, si1, acc,
             in0, in1, sc0, sc1):
    cid = lax.axis_index("c")
    sid = lax.axis_index("s")
    wid = cid * NS + sid
    base_row = wid * ROWS_PER_TILE
    fb = (fb0, fb1)
    wb = (wb0, wb1)
    ib = (ib0, ib1)
    wf = (wf0, wf1)
    si = (si0, si1)
    sin = (in0, in1)
    ssc = (sc0, sc1)

    def issue_inputs(k, p):
        r0 = base_row + k * CHUNK
        pltpu.async_copy(f_hbm.at[pl.ds(r0, CHUNK), :], fb[p], sin[p])
        pltpu.async_copy(w_hbm.at[pl.ds(r0, CHUNK)], wb[p], sin[p])
        pltpu.async_copy(i_hbm.at[pl.ds(r0, CHUNK)], ib[p], sin[p])

    def wait_inputs(k, p):
        r0 = base_row + k * CHUNK
        pltpu.make_async_copy(f_hbm.at[pl.ds(r0, CHUNK), :], fb[p], sin[p]).wait()
        pltpu.make_async_copy(w_hbm.at[pl.ds(r0, CHUNK)], wb[p], sin[p]).wait()
        pltpu.make_async_copy(i_hbm.at[pl.ds(r0, CHUNK)], ib[p], sin[p]).wait()

    def compute(p):
        def _group(g, c2):
            w16 = wb[p][pl.ds(g * 16, 16)]
            si[p][pl.ds(g * 16, 16)] = ib[p][pl.ds(g * 16, 16)]
            for r in range(16):
                row = g * 16 + r
                wbc = jnp.full((16,), w16[r], jnp.float32)
                for c in range(D // 16):
                    wf[p][row, pl.ds(c * 16, 16)] = (
                        fb[p][row, pl.ds(c * 16, 16)] * wbc
                    )
            return c2

        lax.fori_loop(0, CHUNK // 16, _group, 0)

    def issue_scatter(p):
        pltpu.async_copy(wf[p], acc.at[si[p]], ssc[p], add=True)

    def wait_scatter(p):
        pltpu.make_async_copy(wf[p], acc.at[si[p]], ssc[p]).wait()

    issue_inputs(0, 0)

    def _zero_row(r, carry):
        for c in range(D // 16):
            wf0[r, pl.ds(c * 16, 16)] = jnp.zeros((16,), jnp.float32)
        return carry

    lax.fori_loop(0, CHUNK, _zero_row, 0)
    for k in range(SPAD // NS // CHUNK):
        pltpu.sync_copy(
            wf0, acc.at[pl.ds(sid * (SPAD // NS) + k * CHUNK, CHUNK), :]
        )
    plsc.subcore_barrier()

    issue_inputs(1, 1)
    wait_inputs(0, 0)
    compute(0)
    issue_scatter(0)
    issue_inputs(2, 0)
    wait_inputs(1, 1)
    compute(1)
    issue_scatter(1)

    def _pair(m, carry):
        k0 = 2 * m
        issue_inputs(k0 + 1, 1)
        wait_inputs(k0, 0)
        wait_scatter(0)
        compute(0)
        issue_scatter(0)
        issue_inputs(k0 + 2, 0)
        wait_inputs(k0 + 1, 1)
        wait_scatter(1)
        compute(1)
        issue_scatter(1)
        return carry

    lax.fori_loop(1, (NCHUNK - 1) // 2, _pair, 0)

    wait_inputs(NCHUNK - 1, 0)
    wait_scatter(0)
    compute(0)
    issue_scatter(0)
    wait_scatter(1)
    wait_scatter(0)
    plsc.subcore_barrier()

    rows = SPAD // NS
    pltpu.sync_copy(
        acc.at[pl.ds(sid * rows, rows), :],
        out_hbm.at[cid, pl.ds(sid * rows, rows), :],
    )


def _sc_scatter(node_features, wflat, idx):
    mesh = plsc.VectorSubcoreMesh(core_axis_name="c", subcore_axis_name="s")
    run = pl.kernel(
        _sc_body,
        out_type=jax.ShapeDtypeStruct((NC, SPAD, D), jnp.float32),
        mesh=mesh,
        scratch_types=[
            pltpu.VMEM((CHUNK, D), jnp.float32),
            pltpu.VMEM((CHUNK, D), jnp.float32),
            pltpu.VMEM((CHUNK,), jnp.float32),
            pltpu.VMEM((CHUNK,), jnp.float32),
            pltpu.VMEM((CHUNK,), jnp.int32),
            pltpu.VMEM((CHUNK,), jnp.int32),
            pltpu.VMEM((CHUNK, D), jnp.float32),
            pltpu.VMEM((CHUNK, D), jnp.float32),
            pltpu.VMEM((CHUNK,), jnp.int32),
            pltpu.VMEM((CHUNK,), jnp.int32),
            pltpu.VMEM_SHARED((SPAD, D), jnp.float32),
            pltpu.SemaphoreType.DMA,
            pltpu.SemaphoreType.DMA,
            pltpu.SemaphoreType.DMA,
            pltpu.SemaphoreType.DMA,
        ],
    )
    return run(node_features, wflat, idx)


def _combine_body(p_ref, den_ref, o_ref):
    s = p_ref[0] + p_ref[1]
    den = den_ref[...]
    den = jnp.where(den == 0.0, 1.0, den)
    o_ref[...] = s / den


def _combine(parts, den):
    rb = 400
    return pl.pallas_call(
        _combine_body,
        grid=(S // rb,),
        in_specs=[
            pl.BlockSpec((NC, rb, D), lambda i: (0, i, 0)),
            pl.BlockSpec((rb, 1), lambda i: (i, 0)),
        ],
        out_specs=pl.BlockSpec((rb, D), lambda i: (i, 0)),
        out_shape=jax.ShapeDtypeStruct((S, D), jnp.float32),
    )(parts, den)


def kernel(node_features, ensemble_index, W0, b0, W1, b1, W2, b2):
    idx = ensemble_index.astype(jnp.int32)
    b0r = b0.reshape(1, H)
    b1r = b1.reshape(1, H)
    b2r = b2.reshape(1, 1)
    bases = idx[::MLP_B]
    lasts = idx[MLP_B - 1 :: MLP_B]
    nch = (lasts - bases) // DEN_W + 1
    idx3 = idx.reshape(N // MLP_B, 1, MLP_B)
    w2d, den = _mlp_weights(node_features, W0, b0r, W1, b1r, W2, b2r,
                            idx3, bases, nch)
    wflat = w2d.reshape(N)
    parts = _sc_scatter(node_features, wflat, idx)
    return _combine(parts, den)
